# baseline (speedup 1.0000x reference)
import functools

import jax
import jax.numpy as jnp
from jax import lax
from jax.experimental import pallas as pl
from jax.experimental.pallas import tpu as pltpu
from jax.experimental.pallas import tpu_sc as plsc

_F32 = jnp.float32
_DN = (((1,), (1,)), ((), ()))
_DNS = (((1,), (0,)), ((), ()))
_PREC = lax.Precision.DEFAULT


def _sc_gather(idx, table):
    B = idx.shape[0]
    D = table.shape[1]
    info = plsc.get_sparse_core_info()
    nc = info.num_cores
    n_workers = 16
    b_per_w = B // n_workers
    mesh = plsc.VectorSubcoreMesh(core_axis_name="c", subcore_axis_name="s")

    @functools.partial(
        pl.kernel,
        mesh=mesh,
        out_type=jax.ShapeDtypeStruct((B, D), _F32),
        scratch_types=[
            pltpu.VMEM((b_per_w,), jnp.int32),
            pltpu.VMEM((b_per_w, D), _F32),
            pltpu.SemaphoreType.DMA,
        ],
    )
    def gather_kernel(idx_hbm, table_hbm, out_hbm, idx_v, rows_v, sem):
        wid = lax.axis_index("s") * nc + lax.axis_index("c")

        @pl.when(wid < n_workers)
        def _():
            base = wid * b_per_w
            pltpu.sync_copy(idx_hbm.at[pl.ds(base, b_per_w)], idx_v)
            pltpu.async_copy(table_hbm.at[idx_v], rows_v, sem).wait()
            pltpu.sync_copy(rows_v, out_hbm.at[pl.ds(base, b_per_w)])

    return gather_kernel(idx, table)


def _unroll4(body):
    def outer(i, h):
        t = i * 4
        for j in range(4):
            h = body(t + j, h)
        return h
    return outer


def _tc_body(cw_ref, wcat1_ref, uzr1_ref, u1_ref, bias1_ref,
             wcat2_ref, uzr2_ref, u2_ref, bias2_ref,
             wg_ref, bg_ref, out_ref, a_ref, h1_ref, h2_ref):
    H = 512
    T = cw_ref.shape[0]
    bf16 = jnp.bfloat16

    a_ref[...] = lax.dot_general(cw_ref[...], wcat1_ref[...], _DNS,
                                 precision=_PREC) + bias1_ref[...]
    uzr1 = uzr1_ref[...]
    u1 = u1_ref[...]

    def step1(t, h):
        arow = a_ref[pl.ds(t, 1), :]
        hb = h.astype(bf16)
        zr = jax.nn.sigmoid(
            lax.dot_general(hb, uzr1, _DNS, preferred_element_type=_F32)
            + arow[:, :2 * H])
        z = zr[:, :H]
        r = zr[:, H:]
        ht = jnp.tanh(
            lax.dot_general((r * h).astype(bf16), u1, _DNS,
                            preferred_element_type=_F32) + arow[:, 2 * H:])
        hn = h + z * (ht - h)
        h1_ref[pl.ds(t, 1), :] = hn
        return hn

    h0 = jnp.zeros((1, H), _F32)
    lax.fori_loop(0, T // 4, _unroll4(step1), h0)

    a_ref[...] = lax.dot_general(h1_ref[...], wcat2_ref[...], _DNS,
                                 precision=_PREC) + bias2_ref[...]
    uzr2 = uzr2_ref[...]
    u2 = u2_ref[...]

    def step2(t, h):
        arow = a_ref[pl.ds(t, 1), :]
        hb = h.astype(bf16)
        zr = jax.nn.sigmoid(
            lax.dot_general(hb, uzr2, _DNS, preferred_element_type=_F32)
            + arow[:, :2 * H])
        z = zr[:, :H]
        r = zr[:, H:]
        ht = jnp.tanh(
            lax.dot_general((r * h).astype(bf16), u2, _DNS,
                            preferred_element_type=_F32) + arow[:, 2 * H:])
        hn = h + z * (ht - h)
        h2_ref[pl.ds(t, 1), :] = hn
        return hn

    lax.fori_loop(0, T // 4, _unroll4(step2), h0)

    logits = lax.dot_general(h2_ref[...], wg_ref[...], _DN,
                             precision=_PREC) + bg_ref[...]
    m = jnp.max(logits, axis=1, keepdims=True)
    lse = jnp.log(jnp.sum(jnp.exp(logits - m), axis=1, keepdims=True))
    out_ref[...] = logits - m - lse


def kernel(batchinput_tensor, X, W_z_1, U_z_1, W_r_1, U_r_1, W_1, b_W_1,
           U_1, b_U_1, W_z_2, U_z_2, W_r_2, U_r_2, W_2, b_W_2, U_2, b_U_2,
           W_glob, b_glob):
    B, S = batchinput_tensor.shape[0], batchinput_tensor.shape[1]
    T = B * S
    H = U_1.shape[0]
    V = W_glob.shape[0]

    idx = batchinput_tensor[:, :, 0].reshape(-1)
    cw = _sc_gather(idx, X)

    wcat1 = jnp.concatenate([W_z_1.T, W_r_1.T, W_1.T], axis=1)
    wcat2 = jnp.concatenate([W_z_2.T, W_r_2.T, W_2.T], axis=1)
    bf16 = jnp.bfloat16
    uzr1 = jnp.concatenate([U_z_1.T, U_r_1.T], axis=1).astype(bf16)
    uzr2 = jnp.concatenate([U_z_2.T, U_r_2.T], axis=1).astype(bf16)
    zeros2h = jnp.zeros((2 * H,), _F32)
    bias1 = jnp.concatenate([zeros2h, b_W_1 + b_U_1])[None, :]
    bias2 = jnp.concatenate([zeros2h, b_W_2 + b_U_2])[None, :]

    preds = pl.pallas_call(
        _tc_body,
        out_shape=jax.ShapeDtypeStruct((T, V), _F32),
        scratch_shapes=[
            pltpu.VMEM((T, 3 * H), _F32),
            pltpu.VMEM((T, H), _F32),
            pltpu.VMEM((T, H), _F32),
        ],
        compiler_params=pltpu.CompilerParams(
            vmem_limit_bytes=120 * 1024 * 1024,
        ),
    )(cw, wcat1, uzr1, U_1.T.astype(bf16), bias1,
      wcat2, uzr2, U_2.T.astype(bf16), bias2,
      W_glob, b_glob[None, :])

    return preds, jnp.zeros((T,), jnp.int32)
